# baseline (speedup 1.0000x reference)
import functools

import jax
import jax.numpy as jnp
from jax import lax
from jax.experimental import pallas as pl
from jax.experimental.pallas import tpu as pltpu
from jax.experimental.pallas import tpu_sc as plsc

_NC = 2
_NS = 16
_NW = _NC * _NS

_B = 1024
_S = 200
_D = 64
_LW = 2 * _D
_ROWS = _B * _S
_RPW = _ROWS // _NW
_IDXW = 128
_NSTREAM = _RPW // _IDXW
_CH_STREAMS = 5
_CH_ROWS = _CH_STREAMS * _IDXW
_NCH = _NSTREAM // _CH_STREAMS

_EPS = 1e-5


def _sc_gather(lines, idx3d):
    mesh = plsc.VectorSubcoreMesh(core_axis_name="c", subcore_axis_name="s")

    @functools.partial(
        pl.kernel,
        mesh=mesh,
        out_type=jax.ShapeDtypeStruct((_ROWS, _LW), jnp.float32),
        scratch_types=[
            pltpu.VMEM((_NSTREAM, _IDXW), jnp.int32),
            pltpu.VMEM((_CH_ROWS, _LW), jnp.float32),
            pltpu.SemaphoreType.DMA,
        ],
    )
    def k(tab_hbm, idx_hbm, out_hbm, idx_v, rows_v, sem):
        wid = lax.axis_index("s") * _NC + lax.axis_index("c")
        pltpu.sync_copy(idx_hbm.at[wid], idx_v)
        base = wid * _RPW
        for g in range(_NCH):
            handles = []
            for j in range(_CH_STREAMS):
                handles.append(pltpu.async_copy(
                    tab_hbm.at[idx_v.at[g * _CH_STREAMS + j]],
                    rows_v.at[pl.ds(j * _IDXW, _IDXW)],
                    sem,
                ))
            for h in handles:
                h.wait()
            pltpu.sync_copy(
                rows_v, out_hbm.at[pl.ds(base + g * _CH_ROWS, _CH_ROWS)])

    return k(lines, idx3d)


_BB = 8
_BLK = _BB * _S


def _ln_block(x_ref, p_ref, pe_ref, g_ref, b_ref, o_ref):
    xl = x_ref[:, :_D]
    xr = x_ref[:, _D:]
    p = p_ref[...]
    x = xl + (xr - xl) * p + pe_ref[...]
    m = jnp.mean(x, axis=-1, keepdims=True)
    c = x - m
    v = jnp.mean(c * c, axis=-1, keepdims=True)
    y = c * lax.rsqrt(v + _EPS)
    o_ref[...] = y * g_ref[...] + b_ref[...]


def _tc_layernorm(gathered, parity, pe_tile, gamma2, beta2):
    grid = _ROWS // _BLK
    return pl.pallas_call(
        _ln_block,
        grid=(grid,),
        in_specs=[
            pl.BlockSpec((_BLK, _LW), lambda i: (i, 0)),
            pl.BlockSpec((_BLK, 1), lambda i: (i, 0)),
            pl.BlockSpec((_BLK, _D), lambda i: (0, 0)),
            pl.BlockSpec((1, _D), lambda i: (0, 0)),
            pl.BlockSpec((1, _D), lambda i: (0, 0)),
        ],
        out_specs=pl.BlockSpec((_BLK, _D), lambda i: (i, 0)),
        out_shape=jax.ShapeDtypeStruct((_ROWS, _D), jnp.float32),
    )(gathered, parity, pe_tile, gamma2, beta2)


def kernel(input_ids, table, gamma, beta, pos_enc):
    ids = input_ids.reshape(-1).astype(jnp.int32)
    lines = table.reshape(table.shape[0] // 2, _LW)
    lidx3d = (ids >> 1).reshape(_NW, _NSTREAM, _IDXW)
    parity = (ids & 1).astype(jnp.float32).reshape(_ROWS, 1)
    gathered = _sc_gather(lines, lidx3d)
    pe = pos_enc[0, :_S, :]
    pe_tile = jnp.tile(pe, (_BB, 1))
    out = _tc_layernorm(gathered, parity, pe_tile,
                        gamma.reshape(1, _D), beta.reshape(1, _D))
    return out.reshape(_B, _S, _D)


# speedup vs baseline: 1.1781x; 1.1781x over previous
import functools

import jax
import jax.numpy as jnp
from jax import lax
from jax.experimental import pallas as pl
from jax.experimental.pallas import tpu as pltpu
from jax.experimental.pallas import tpu_sc as plsc

_NC = 2
_NS = 16
_NW = _NC * _NS

_B = 1024
_S = 200
_D = 64
_ROWS = _B * _S
_RPW = _ROWS // _NW
_IDXW = 128
_NSTREAM = _RPW // _IDXW
_CH_STREAMS = 10
_CH_ROWS = _CH_STREAMS * _IDXW
_NCH = _NSTREAM // _CH_STREAMS

_EPS = 1e-5


def _sc_gather(table, idx3d):
    mesh = plsc.VectorSubcoreMesh(core_axis_name="c", subcore_axis_name="s")

    @functools.partial(
        pl.kernel,
        mesh=mesh,
        compiler_params=pltpu.CompilerParams(use_tc_tiling_on_sc=False),
        out_type=jax.ShapeDtypeStruct((_ROWS, _D), jnp.float32),
        scratch_types=[
            pltpu.VMEM((_NSTREAM, _IDXW), jnp.int32),
            pltpu.VMEM((_CH_ROWS, _D), jnp.float32),
            pltpu.SemaphoreType.DMA,
        ],
    )
    def k(tab_hbm, idx_hbm, out_hbm, idx_v, rows_v, sem):
        wid = lax.axis_index("s") * _NC + lax.axis_index("c")
        pltpu.sync_copy(idx_hbm.at[wid], idx_v)
        base = wid * _RPW
        for g in range(_NCH):
            handles = []
            for j in range(_CH_STREAMS):
                handles.append(pltpu.async_copy(
                    tab_hbm.at[idx_v.at[g * _CH_STREAMS + j]],
                    rows_v.at[pl.ds(j * _IDXW, _IDXW)],
                    sem,
                ))
            for h in handles:
                h.wait()
            pltpu.sync_copy(
                rows_v, out_hbm.at[pl.ds(base + g * _CH_ROWS, _CH_ROWS)])

    return k(table, idx3d)


_SB = 8


def _ln_block(x_ref, pe_ref, g_ref, b_ref, o_ref):
    x = x_ref[...] + pe_ref[...]
    m = jnp.mean(x, axis=1, keepdims=True)
    c = x - m
    v = jnp.mean(c * c, axis=1, keepdims=True)
    y = c * lax.rsqrt(v + _EPS)
    o_ref[...] = y * g_ref[...] + b_ref[...]


def _tc_layernorm(xt, pe_t, gamma_t, beta_t):
    grid = _S // _SB
    return pl.pallas_call(
        _ln_block,
        grid=(grid,),
        in_specs=[
            pl.BlockSpec((_SB, _D, _B), lambda i: (i, 0, 0)),
            pl.BlockSpec((_SB, _D, 1), lambda i: (i, 0, 0)),
            pl.BlockSpec((1, _D, 1), lambda i: (0, 0, 0)),
            pl.BlockSpec((1, _D, 1), lambda i: (0, 0, 0)),
        ],
        out_specs=pl.BlockSpec((_SB, _D, _B), lambda i: (i, 0, 0)),
        out_shape=jax.ShapeDtypeStruct((_S, _D, _B), jnp.float32),
    )(xt, pe_t, gamma_t, beta_t)


def kernel(input_ids, table, gamma, beta, pos_enc):
    ids = input_ids.reshape(-1).astype(jnp.int32)
    idx3d = ids.reshape(_NW, _NSTREAM, _IDXW)
    gathered = _sc_gather(table, idx3d)
    xt = gathered.reshape(_B, _S, _D).transpose(1, 2, 0)
    pe_t = pos_enc[0, :_S, :].reshape(_S, _D, 1)
    out_t = _tc_layernorm(xt, pe_t,
                          gamma.reshape(1, _D, 1), beta.reshape(1, _D, 1))
    return out_t.transpose(2, 0, 1)


# speedup vs baseline: 1.7458x; 1.4818x over previous
import functools

import jax
import jax.numpy as jnp
from jax import lax
from jax.experimental import pallas as pl
from jax.experimental.pallas import tpu as pltpu
from jax.experimental.pallas import tpu_sc as plsc

_NC = 2
_NS = 16
_NW = _NC * _NS

_B = 1024
_S = 200
_D = 64
_ROWS = _B * _S
_RPW = _ROWS // _NW
_IDXW = 128
_NSTREAM = _RPW // _IDXW
_CH_STREAMS = 10
_CH_ROWS = _CH_STREAMS * _IDXW
_NCH = _NSTREAM // _CH_STREAMS

_EPS = 1e-5


def _sc_gather(table, idx3d):
    mesh = plsc.VectorSubcoreMesh(core_axis_name="c", subcore_axis_name="s")

    @functools.partial(
        pl.kernel,
        mesh=mesh,
        compiler_params=pltpu.CompilerParams(use_tc_tiling_on_sc=False),
        out_type=jax.ShapeDtypeStruct((_ROWS, _D), jnp.float32),
        scratch_types=[
            pltpu.VMEM((_NSTREAM, _IDXW), jnp.int32),
            pltpu.VMEM((_CH_ROWS, _D), jnp.float32),
            pltpu.SemaphoreType.DMA,
        ],
    )
    def k(tab_hbm, idx_hbm, out_hbm, idx_v, rows_v, sem):
        wid = lax.axis_index("s") * _NC + lax.axis_index("c")
        pltpu.sync_copy(idx_hbm.at[wid], idx_v)
        base = wid * _RPW
        for g in range(_NCH):
            handles = []
            for j in range(_CH_STREAMS):
                handles.append(pltpu.async_copy(
                    tab_hbm.at[idx_v.at[g * _CH_STREAMS + j]],
                    rows_v.at[pl.ds(j * _IDXW, _IDXW)],
                    sem,
                ))
            for h in handles:
                h.wait()
            pltpu.sync_copy(
                rows_v, out_hbm.at[pl.ds(base + g * _CH_ROWS, _CH_ROWS)])

    return k(table, idx3d)


_TCOL = 2048
_TGRID = 248
_SPLIT = _TGRID * _TCOL
_VLAST = -(-1000000 // _TCOL) - 1


def _tr_block(xa_ref, xb_ref, o_ref):
    o_ref[:, :_D] = jnp.swapaxes(xa_ref[...], 0, 1)
    o_ref[:, _D:] = jnp.swapaxes(xb_ref[...], 0, 1)


def _tc_transpose(table_t):
    return pl.pallas_call(
        _tr_block,
        grid=(_TGRID,),
        in_specs=[
            pl.BlockSpec((_D, _TCOL), lambda i: (0, i)),
            pl.BlockSpec((_D, _TCOL),
                         lambda i: (0, jnp.minimum(i + _TGRID, _VLAST))),
        ],
        out_specs=pl.BlockSpec((_TCOL, 128), lambda i: (i, 0)),
        out_shape=jax.ShapeDtypeStruct((_SPLIT, 128), jnp.float32),
    )(table_t, table_t)


_SB = 8


def _ln_block(x_ref, pe_ref, g_ref, b_ref, o_ref):
    x = x_ref[...] + pe_ref[...]
    m = jnp.mean(x, axis=1, keepdims=True)
    c = x - m
    v = jnp.mean(c * c, axis=1, keepdims=True)
    y = c * lax.rsqrt(v + _EPS)
    o_ref[...] = y * g_ref[...] + b_ref[...]


def _tc_layernorm(xt, pe_t, gamma_t, beta_t):
    grid = _S // _SB
    return pl.pallas_call(
        _ln_block,
        grid=(grid,),
        in_specs=[
            pl.BlockSpec((_SB, _D, _B), lambda i: (i, 0, 0)),
            pl.BlockSpec((_SB, _D, 1), lambda i: (i, 0, 0)),
            pl.BlockSpec((1, _D, 1), lambda i: (0, 0, 0)),
            pl.BlockSpec((1, _D, 1), lambda i: (0, 0, 0)),
        ],
        out_specs=pl.BlockSpec((_SB, _D, _B), lambda i: (i, 0, 0)),
        out_shape=jax.ShapeDtypeStruct((_S, _D, _B), jnp.float32),
    )(xt, pe_t, gamma_t, beta_t)


def kernel(input_ids, table, gamma, beta, pos_enc):
    ids = input_ids.reshape(-1).astype(jnp.int32)
    gidx = jnp.where(ids < _SPLIT, ids * 2, (ids - _SPLIT) * 2 + 1)
    idx3d = gidx.reshape(_NW, _NSTREAM, _IDXW)
    lines = _tc_transpose(table.T)
    tab_lin = lines.reshape(-1).reshape(2 * _SPLIT, _D)
    gathered = _sc_gather(tab_lin, idx3d)
    xt = gathered.reshape(_B, _S, _D).transpose(1, 2, 0)
    pe_t = pos_enc[0, :_S, :].reshape(_S, _D, 1)
    out_t = _tc_layernorm(xt, pe_t,
                          gamma.reshape(1, _D, 1), beta.reshape(1, _D, 1))
    return out_t.transpose(2, 0, 1)


# speedup vs baseline: 1.9949x; 1.1427x over previous
import functools

import jax
import jax.numpy as jnp
from jax import lax
from jax.experimental import pallas as pl
from jax.experimental.pallas import tpu as pltpu
from jax.experimental.pallas import tpu_sc as plsc

_NC = 2
_NS = 16
_NW = _NC * _NS

_B = 1024
_S = 200
_D = 64
_ROWS = _B * _S
_RPW = _ROWS // _NW
_IDXW = 128
_NSTREAM = _RPW // _IDXW
_CH_STREAMS = 10
_CH_ROWS = _CH_STREAMS * _IDXW
_NCH = _NSTREAM // _CH_STREAMS

_EPS = 1e-5


def _sc_gather(table, idx3d):
    mesh = plsc.VectorSubcoreMesh(core_axis_name="c", subcore_axis_name="s")

    @functools.partial(
        pl.kernel,
        mesh=mesh,
        compiler_params=pltpu.CompilerParams(use_tc_tiling_on_sc=False),
        out_type=jax.ShapeDtypeStruct((_ROWS, _D), jnp.float32),
        scratch_types=[
            pltpu.VMEM((_NSTREAM, _IDXW), jnp.int32),
            pltpu.VMEM((_CH_ROWS, _D), jnp.float32),
            pltpu.SemaphoreType.DMA,
        ],
    )
    def k(tab_hbm, idx_hbm, out_hbm, idx_v, rows_v, sem):
        wid = lax.axis_index("s") * _NC + lax.axis_index("c")
        pltpu.sync_copy(idx_hbm.at[wid], idx_v)
        base = wid * _RPW
        for g in range(_NCH):
            handles = []
            for j in range(_CH_STREAMS):
                handles.append(pltpu.async_copy(
                    tab_hbm.at[idx_v.at[g * _CH_STREAMS + j]],
                    rows_v.at[pl.ds(j * _IDXW, _IDXW)],
                    sem,
                ))
            for h in handles:
                h.wait()
            pltpu.sync_copy(
                rows_v, out_hbm.at[pl.ds(base + g * _CH_ROWS, _CH_ROWS)])

    return k(table, idx3d)


_TCOL = 4096
_TGRID = 124
_SPLIT = _TGRID * _TCOL
_VLAST = -(-1000000 // _TCOL) - 1


def _tr_block(xa_ref, xb_ref, o_ref):
    o_ref[:, :_D] = jnp.swapaxes(xa_ref[...], 0, 1)
    o_ref[:, _D:] = jnp.swapaxes(xb_ref[...], 0, 1)


def _tc_transpose(table_t):
    return pl.pallas_call(
        _tr_block,
        grid=(_TGRID,),
        in_specs=[
            pl.BlockSpec((_D, _TCOL), lambda i: (0, i)),
            pl.BlockSpec((_D, _TCOL),
                         lambda i: (0, jnp.minimum(i + _TGRID, _VLAST))),
        ],
        out_specs=pl.BlockSpec((_TCOL, 128), lambda i: (i, 0)),
        out_shape=jax.ShapeDtypeStruct((_SPLIT, 128), jnp.float32),
    )(table_t, table_t)


_SB = 8


def _ln_block(x_ref, pe_ref, g_ref, b_ref, o_ref):
    x = x_ref[...] + pe_ref[...]
    m = jnp.mean(x, axis=1, keepdims=True)
    c = x - m
    v = jnp.mean(c * c, axis=1, keepdims=True)
    y = c * lax.rsqrt(v + _EPS)
    o_ref[...] = y * g_ref[...] + b_ref[...]


def _tc_layernorm(xt, pe_t, gamma_t, beta_t):
    grid = _S // _SB
    return pl.pallas_call(
        _ln_block,
        grid=(grid,),
        in_specs=[
            pl.BlockSpec((_SB, _D, _B), lambda i: (i, 0, 0)),
            pl.BlockSpec((_SB, _D, 1), lambda i: (i, 0, 0)),
            pl.BlockSpec((1, _D, 1), lambda i: (0, 0, 0)),
            pl.BlockSpec((1, _D, 1), lambda i: (0, 0, 0)),
        ],
        out_specs=pl.BlockSpec((_SB, _D, _B), lambda i: (i, 0, 0)),
        out_shape=jax.ShapeDtypeStruct((_S, _D, _B), jnp.float32),
    )(xt, pe_t, gamma_t, beta_t)


def kernel(input_ids, table, gamma, beta, pos_enc):
    ids = input_ids.reshape(-1).astype(jnp.int32)
    gidx = jnp.where(ids < _SPLIT, ids * 2, (ids - _SPLIT) * 2 + 1)
    idx3d = gidx.reshape(_NW, _NSTREAM, _IDXW)
    lines = _tc_transpose(table.T)
    tab_lin = lines.reshape(-1).reshape(2 * _SPLIT, _D)
    gathered = _sc_gather(tab_lin, idx3d)
    xt = gathered.reshape(_B, _S, _D).transpose(1, 2, 0)
    pe_t = pos_enc[0, :_S, :].reshape(_S, _D, 1)
    out_t = _tc_layernorm(xt, pe_t,
                          gamma.reshape(1, _D, 1), beta.reshape(1, _D, 1))
    return out_t.transpose(2, 0, 1)
